# baseline (speedup 1.0000x reference)
import functools

import jax
import jax.numpy as jnp
from jax import lax
from jax.experimental import pallas as pl
from jax.experimental.pallas import tpu as pltpu
from jax.experimental.pallas import tpu_sc as plsc

MAXLEN = 200
DIM = 64
LANES = 16
NC, NS = 2, 16
NW = NC * NS
NBUF = 4
ROUNDS = MAXLEN // NBUF


def _pe_table():
    position = jnp.arange(MAXLEN, dtype=jnp.float32)[:, None]
    div_term = jnp.exp(
        jnp.arange(0, DIM, 2, dtype=jnp.float32) * (-jnp.log(10000.0) / DIM)
    )
    pe = jnp.zeros((MAXLEN, (DIM + 1) // 2 * 2), dtype=jnp.float32)
    pe = pe.at[:, 0::2].set(jnp.sin(position * div_term))
    pe = pe.at[:, 1::2].set(jnp.cos(position * div_term))
    return pe[:, :DIM]


def _make_sc_call(batch):
    bblk = batch // NW
    mesh = plsc.VectorSubcoreMesh(core_axis_name="c", subcore_axis_name="s")

    @functools.partial(
        pl.kernel,
        out_type=jax.ShapeDtypeStruct((batch, MAXLEN, DIM), jnp.float32),
        mesh=mesh,
        scratch_types=[
            pltpu.VMEM((bblk * MAXLEN,), jnp.int32),
            pltpu.VMEM((MAXLEN, bblk), jnp.int32),
            pltpu.VMEM((MAXLEN, DIM), jnp.float32),
            [pltpu.VMEM((bblk, DIM), jnp.float32) for _ in range(NBUF)],
            [pltpu.SemaphoreType.DMA for _ in range(NBUF)],
            [pltpu.SemaphoreType.DMA for _ in range(NBUF)],
        ],
        compiler_params=pltpu.CompilerParams(
            use_tc_tiling_on_sc=False, needs_layout_passes=False
        ),
    )
    def sc_kernel(x_hbm, w_hbm, pe_hbm, out_hbm, xb_v, idx_v, pe_v, rows, gsem, ssem):
        wid = lax.axis_index("s") * NC + lax.axis_index("c")
        b0 = wid * bblk
        pltpu.sync_copy(x_hbm.at[pl.ds(b0 * MAXLEN, bblk * MAXLEN)], xb_v)
        pltpu.sync_copy(pe_hbm, pe_v)

        lane = lax.iota(jnp.int32, LANES)
        rvecs = [(lane + m * LANES) * MAXLEN for m in range(bblk // LANES)]

        @pl.loop(0, MAXLEN)
        def _tr(t):
            for m in range(bblk // LANES):
                idx_v[t, pl.ds(m * LANES, LANES)] = plsc.load_gather(
                    xb_v, [rvecs[m] + t]
                )

        @pl.loop(0, ROUNDS)
        def _round(g):
            gdesc = []
            for k in range(NBUF):
                t = g * NBUF + k

                @pl.when(g > 0)
                def _():
                    pltpu.make_async_copy(
                        rows[k], out_hbm.at[pl.ds(b0, bblk), 0], ssem[k]
                    ).wait()

                gdesc.append(
                    pltpu.async_copy(w_hbm.at[idx_v.at[t]], rows[k], gsem[k])
                )
            for k in range(NBUF):
                t = g * NBUF + k
                gdesc[k].wait()
                pvec = [pe_v[t, pl.ds(j * LANES, LANES)] for j in range(DIM // LANES)]

                @pl.loop(0, bblk, unroll=8)
                def _row(r):
                    for j in range(DIM // LANES):
                        plsc.addupdate(rows[k].at[r, pl.ds(j * LANES, LANES)], pvec[j])

                pltpu.async_copy(
                    rows[k], out_hbm.at[pl.ds(b0, bblk), t], ssem[k]
                )

        for k in range(NBUF):
            pltpu.make_async_copy(
                rows[k], out_hbm.at[pl.ds(b0, bblk), 0], ssem[k]
            ).wait()

    return sc_kernel


def kernel(x, W):
    b, _ = x.shape
    pe = _pe_table()
    return _make_sc_call(b)(x.reshape(-1), W, pe)


# speedup vs baseline: 1.0058x; 1.0058x over previous
import functools

import jax
import jax.numpy as jnp
from jax import lax
from jax.experimental import pallas as pl
from jax.experimental.pallas import tpu as pltpu
from jax.experimental.pallas import tpu_sc as plsc

MAXLEN = 200
DIM = 64
LANES = 16
NC, NS = 2, 16
NW = NC * NS
NBUF = 4
ROUNDS = MAXLEN // NBUF
SUB = 8


def _pe_table():
    position = jnp.arange(MAXLEN, dtype=jnp.float32)[:, None]
    div_term = jnp.exp(
        jnp.arange(0, DIM, 2, dtype=jnp.float32) * (-jnp.log(10000.0) / DIM)
    )
    pe = jnp.zeros((MAXLEN, (DIM + 1) // 2 * 2), dtype=jnp.float32)
    pe = pe.at[:, 0::2].set(jnp.sin(position * div_term))
    pe = pe.at[:, 1::2].set(jnp.cos(position * div_term))
    return pe[:, :DIM]


def _make_sc_call(batch):
    bblk = batch // NW
    tt_n = MAXLEN // SUB
    mesh = plsc.VectorSubcoreMesh(core_axis_name="c", subcore_axis_name="s")

    @functools.partial(
        pl.kernel,
        out_type=jax.ShapeDtypeStruct((batch, MAXLEN, DIM), jnp.float32),
        mesh=mesh,
        scratch_types=[
            pltpu.VMEM((tt_n, SUB, bblk), jnp.int32),
            pltpu.VMEM((MAXLEN, DIM), jnp.float32),
            [pltpu.VMEM((bblk, DIM), jnp.float32) for _ in range(NBUF)],
            [pltpu.SemaphoreType.DMA for _ in range(NBUF)],
            [pltpu.SemaphoreType.DMA for _ in range(NBUF)],
        ],
        compiler_params=pltpu.CompilerParams(
            use_tc_tiling_on_sc=False, needs_layout_passes=False
        ),
    )
    def sc_kernel(xt_hbm, w_hbm, pe_hbm, out_hbm, idx_v, pe_v, rows, gsem, ssem):
        wid = lax.axis_index("s") * NC + lax.axis_index("c")
        b0 = wid * bblk
        pltpu.sync_copy(xt_hbm.at[:, wid], idx_v)
        pltpu.sync_copy(pe_hbm, pe_v)

        @pl.loop(0, ROUNDS)
        def _round(g):
            gdesc = []
            for k in range(NBUF):
                t = g * NBUF + k

                @pl.when(g > 0)
                def _():
                    pltpu.make_async_copy(
                        rows[k], out_hbm.at[pl.ds(b0, bblk), 0], ssem[k]
                    ).wait()

                gdesc.append(
                    pltpu.async_copy(
                        w_hbm.at[idx_v.at[lax.div(t, SUB), lax.rem(t, SUB)]],
                        rows[k],
                        gsem[k],
                    )
                )
            for k in range(NBUF):
                t = g * NBUF + k
                gdesc[k].wait()
                pvec = [pe_v[t, pl.ds(j * LANES, LANES)] for j in range(DIM // LANES)]

                @pl.loop(0, bblk, unroll=8)
                def _row(r):
                    for j in range(DIM // LANES):
                        plsc.addupdate(rows[k].at[r, pl.ds(j * LANES, LANES)], pvec[j])

                pltpu.async_copy(
                    rows[k], out_hbm.at[pl.ds(b0, bblk), t], ssem[k]
                )

        for k in range(NBUF):
            pltpu.make_async_copy(
                rows[k], out_hbm.at[pl.ds(b0, bblk), 0], ssem[k]
            ).wait()

    return sc_kernel


def kernel(x, W):
    b, t = x.shape
    pe = _pe_table()
    bblk = b // NW
    xt = x.reshape(NW, bblk, t // SUB, SUB).transpose(2, 0, 3, 1)
    return _make_sc_call(b)(xt, W, pe)
